# baseline (speedup 1.0000x reference)
import functools

import jax
import jax.numpy as jnp
from jax import lax
from jax.experimental import pallas as pl
from jax.experimental.pallas import tpu as pltpu
from jax.experimental.pallas import tpu_sc as plsc

N = 10000
E = 320000
D_IN = 128
HID = 32
HEADS = 8
D_OUT = 128

BLK = 1000
NB = N // BLK
NSUB = 16
NCORE = 2
EDGES_PER_SUB = E // NSUB
ESUB = 20224
EB1 = 64
NPAIR1 = ESUB // (2 * EB1)
EB2 = 128
NPAIR2 = ESUB // (2 * EB2)
NSH = N + 8
ROWS_PER_SUB = 640
K1 = 136
K2 = 72

_f32 = jnp.float32
_i32 = jnp.int32

_GDN = lax.GatherDimensionNumbers(
    offset_dims=(), collapsed_slice_dims=(0,), start_index_map=(0,))


def _bcast(vec, idx):
    return lax.gather(vec, idx[:, None], _GDN, (1,),
                      mode=lax.GatherScatterMode.PROMISE_IN_BOUNDS)


def _tc_a(x_ref, w1_ref, asm_ref, adm_ref, haug_ref, t1d_ref, a1_ref):
    i = pl.program_id(0)
    h = jnp.dot(x_ref[...], w1_ref[...], preferred_element_type=_f32)
    als = jnp.dot(h, asm_ref[...], preferred_element_type=_f32)
    ald = jnp.dot(h, adm_ref[...], preferred_element_type=_f32)
    haug_ref[0] = jnp.concatenate([h[:, :128], als], axis=1)
    haug_ref[1] = jnp.concatenate([h[:, 128:], als], axis=1)
    t1d_ref[...] = ald
    cur = jnp.broadcast_to(jnp.max(als, axis=0)[None, :], (8, 8))

    @pl.when(i == 0)
    def _():
        a1_ref[...] = cur

    @pl.when(i > 0)
    def _():
        a1_ref[...] = jnp.maximum(a1_ref[...], cur)


def _call_a(x, w1, asm, adm):
    return pl.pallas_call(
        _tc_a,
        grid=(NB,),
        in_specs=[
            pl.BlockSpec((BLK, D_IN), lambda i: (i, 0)),
            pl.BlockSpec((D_IN, HEADS * HID), lambda i: (0, 0)),
            pl.BlockSpec((HEADS * HID, 8), lambda i: (0, 0)),
            pl.BlockSpec((HEADS * HID, 8), lambda i: (0, 0)),
        ],
        out_specs=[
            pl.BlockSpec((2, BLK, K1), lambda i: (0, i, 0)),
            pl.BlockSpec((BLK, 8), lambda i: (i, 0)),
            pl.BlockSpec((8, 8), lambda i: (0, 0)),
        ],
        out_shape=[
            jax.ShapeDtypeStruct((2, N, K1), _f32),
            jax.ShapeDtypeStruct((N, 8), _f32),
            jax.ShapeDtypeStruct((8, 8), _f32),
        ],
    )(x, w1, asm, adm)


_MESH = plsc.VectorSubcoreMesh(core_axis_name="c", subcore_axis_name="s")


def _zero_msg(msg, EB, K, full):
    zero16 = jnp.zeros((16,), _f32)
    for r in range(EB):
        if full:
            for k in range(K // 16):
                msg[r, pl.ds(k * 16, 16)] = zero16
        msg[r, pl.ds(K - 16, 16)] = zero16


def _zero_stripe(msg, shared, s, base_r, EB):
    n0 = ROWS_PER_SUB // EB
    n15 = 400 // EB
    tail = 400 - n15 * EB

    @pl.when(s < 15)
    def _():
        for j in range(n0):
            pltpu.sync_copy(msg, shared.at[pl.ds(base_r + j * EB, EB)])

    @pl.when(s == 15)
    def _():
        for j in range(n15):
            pltpu.sync_copy(msg, shared.at[pl.ds(base_r + j * EB, EB)])
        if tail:
            pltpu.sync_copy(msg.at[pl.ds(0, tail)],
                            shared.at[pl.ds(base_r + n15 * EB, tail)])


def _copy_out(msg, shared, oa, s, base_r, out_base, EB):
    n0 = ROWS_PER_SUB // EB
    n15 = 400 // EB
    tail = 400 - n15 * EB

    @pl.when(s < 15)
    def _():
        for j in range(n0):
            pltpu.sync_copy(shared.at[pl.ds(base_r + j * EB, EB)], msg)
            pltpu.sync_copy(msg, oa.at[pl.ds(out_base + j * EB, EB)])

    @pl.when(s == 15)
    def _():
        for j in range(n15):
            pltpu.sync_copy(shared.at[pl.ds(base_r + j * EB, EB)], msg)
            pltpu.sync_copy(msg, oa.at[pl.ds(out_base + j * EB, EB)])
        if tail:
            pltpu.sync_copy(shared.at[pl.ds(base_r + n15 * EB, tail)],
                            msg.at[pl.ds(0, tail)])
            pltpu.sync_copy(msg.at[pl.ds(0, tail)],
                            oa.at[pl.ds(out_base + n15 * EB, tail)])


def _make_sc_edge(NH, F, K, EB, NPAIR, as_col_fn, ad_col_fn):
    CPH = F // NH // 16

    def kern(ei, tt, hh, asmv, oa, shared, srcba, srcbb, dstba, dstbb,
             sdsta, sdstb, offba, offbb, tdsta, tdstb, hbufa, hbufb,
             msga, msgb, asmb, isema, isemb, gsema, gsemb, ssema, ssemb):
        c = lax.axis_index("c")
        s = lax.axis_index("s")
        lane = lax.iota(_i32, 16)
        pltpu.sync_copy(asmv.at[c], asmb)

        _zero_msg(msga, EB, K, full=True)
        _zero_msg(msgb, EB, K, full=False)
        base_r = s * ROWS_PER_SUB
        _zero_stripe(msga, shared, s, base_r, EB)
        plsc.subcore_barrier()

        cn = c * N
        asmlv = asmb[...]
        asml = [_bcast(asmlv, jnp.full((16,), h, _i32)) for h in range(NH)]
        cas = [as_col_fn(h, c) for h in range(NH)]
        cad = [ad_col_fn(h, c) for h in range(NH)]

        def enq_idx(bi, srcb, dstb, isem):
            pltpu.async_copy(ei.at[0, s, bi], srcb, isem)
            pltpu.async_copy(ei.at[1, s, bi], dstb, isem)

        def wait_idx(bi, srcb, dstb, isem):
            pltpu.make_async_copy(ei.at[0, s, bi], srcb, isem).wait()
            pltpu.make_async_copy(ei.at[1, s, bi], dstb, isem).wait()

        def fill_off(srcb, offb):
            for j in range(EB // 16):
                offb[pl.ds(j * 16, 16)] = srcb[pl.ds(j * 16, 16)] + cn

        def enq_g(offb, dstb, hbuf, tdst, gsem):
            pltpu.async_copy(hh.at[offb], hbuf, gsem)
            pltpu.async_copy(tt.at[dstb], tdst, gsem)

        def wait_g(offb, dstb, hbuf, tdst, gsem):
            pltpu.make_async_copy(hh.at[offb], hbuf, gsem).wait()
            pltpu.make_async_copy(tt.at[dstb], tdst, gsem).wait()

        def fill_sdst(dstb, sdst):
            for j in range(EB // 16):
                sdst[pl.ds(j * 16, 16)] = dstb[pl.ds(j * 16, 16)]

        def enq_s(msg, sdst, ssem):
            pltpu.async_copy(msg, shared.at[sdst], ssem, add=True)

        def wait_s(msg, sdst, ssem):
            pltpu.make_async_copy(msg, shared.at[sdst], ssem).wait()

        def compute(hbuf, tdst, msg):
            for j in range(EB // 16):
                rowv = lane + j * 16
                ees = []
                for h in range(NH):
                    asv = plsc.load_gather(hbuf, [rowv, cas[h]])
                    adv = plsc.load_gather(tdst, [rowv, cad[h]])
                    sv = asv + adv
                    ev = jnp.where(sv > 0, sv, 0.2 * sv)
                    mr = asml[h] + adv
                    mv = jnp.where(mr > 0, mr, 0.2 * mr)
                    ees.append(jnp.exp(ev - mv))
                for h in range(NH):
                    plsc.store_scatter(
                        msg, [rowv, jnp.full((16,), F + h, _i32)], ees[h])
                for e in range(16):
                    row = j * 16 + e
                    idx = jnp.full((16,), e, _i32)
                    for h in range(NH):
                        b = _bcast(ees[h], idx)
                        for k2 in range(CPH):
                            col = (h * CPH + k2) * 16
                            msg[row, pl.ds(col, 16)] = (
                                hbuf[row, pl.ds(col, 16)] * b)

        enq_idx(0, srcba, dstba, isema)
        enq_idx(1, srcbb, dstbb, isemb)
        wait_idx(0, srcba, dstba, isema)
        fill_off(srcba, offba)
        enq_g(offba, dstba, hbufa, tdsta, gsema)

        def body(i, _):
            ba = 2 * i
            wait_idx(ba + 1, srcbb, dstbb, isemb)
            fill_off(srcbb, offbb)
            enq_g(offbb, dstbb, hbufb, tdstb, gsemb)
            wait_g(offba, dstba, hbufa, tdsta, gsema)

            @pl.when(i > 0)
            def _():
                wait_s(msga, sdsta, ssema)

            fill_sdst(dstba, sdsta)

            @pl.when(i < NPAIR - 1)
            def _():
                enq_idx(ba + 2, srcba, dstba, isema)

            compute(hbufa, tdsta, msga)
            enq_s(msga, sdsta, ssema)

            @pl.when(i < NPAIR - 1)
            def _():
                wait_idx(ba + 2, srcba, dstba, isema)
                fill_off(srcba, offba)
                enq_g(offba, dstba, hbufa, tdsta, gsema)

            wait_g(offbb, dstbb, hbufb, tdstb, gsemb)

            @pl.when(i > 0)
            def _():
                wait_s(msgb, sdstb, ssemb)

            fill_sdst(dstbb, sdstb)

            @pl.when(i < NPAIR - 1)
            def _():
                enq_idx(ba + 3, srcbb, dstbb, isemb)

            compute(hbufb, tdstb, msgb)
            enq_s(msgb, sdstb, ssemb)
            return 0

        lax.fori_loop(0, NPAIR, body, 0)
        wait_s(msga, sdsta, ssema)
        wait_s(msgb, sdstb, ssemb)
        plsc.subcore_barrier()
        _copy_out(msga, shared, oa, s, base_r, cn + base_r, EB)

    return kern


def _call_sc(kern, K, EB, ei4, tt, hh, asmv):
    fn = functools.partial(
        pl.kernel,
        out_type=jax.ShapeDtypeStruct((2 * N, K), _f32),
        mesh=_MESH,
        compiler_params=pltpu.CompilerParams(
            use_tc_tiling_on_sc=False, needs_layout_passes=False),
        scratch_types=[
            pltpu.VMEM_SHARED((NSH, K), _f32),
            pltpu.VMEM((EB,), _i32),
            pltpu.VMEM((EB,), _i32),
            pltpu.VMEM((EB,), _i32),
            pltpu.VMEM((EB,), _i32),
            pltpu.VMEM((EB,), _i32),
            pltpu.VMEM((EB,), _i32),
            pltpu.VMEM((EB,), _i32),
            pltpu.VMEM((EB,), _i32),
            pltpu.VMEM((EB, 8), _f32),
            pltpu.VMEM((EB, 8), _f32),
            pltpu.VMEM((EB, K), _f32),
            pltpu.VMEM((EB, K), _f32),
            pltpu.VMEM((EB, K), _f32),
            pltpu.VMEM((EB, K), _f32),
            pltpu.VMEM((16,), _f32),

            pltpu.SemaphoreType.DMA,
            pltpu.SemaphoreType.DMA,
            pltpu.SemaphoreType.DMA,
            pltpu.SemaphoreType.DMA,
            pltpu.SemaphoreType.DMA,
            pltpu.SemaphoreType.DMA,
        ],
    )(kern)
    return fn(ei4, tt, hh, asmv)


_sc_l1 = _make_sc_edge(
    4, 128, K1, EB1, NPAIR1,
    lambda h, c: jnp.full((16,), 128 + h, _i32) + 4 * c,
    lambda h, c: jnp.full((16,), h, _i32) + 4 * c)

_sc_l2 = _make_sc_edge(
    1, 64, K2, EB2, NPAIR2,
    lambda h, c: jnp.full((16,), 64, _i32),
    lambda h, c: jnp.zeros((16,), _i32))


def _tc_c(oa0_ref, oa1_ref, b1_ref, w2_ref, as2_ref, ad2_ref, r_ref,
          h2p_ref, t2d_ref, a2_ref):
    i = pl.program_id(0)
    f0 = oa0_ref[...]
    f1 = oa1_ref[...]
    r = r_ref[...]
    den0 = jnp.dot(f0[:, 128:132], r, preferred_element_type=_f32)
    den1 = jnp.dot(f1[:, 128:132], r, preferred_element_type=_f32)
    p0 = f0[:, :128] / (den0 + 1e-16)
    p1 = f1[:, :128] / (den1 + 1e-16)
    h1n = jnp.concatenate([p0, p1], axis=1) + b1_ref[...]
    e1 = jnp.where(h1n > 0, h1n, jnp.exp(h1n) - 1.0)
    h2 = jnp.dot(e1, w2_ref[...], preferred_element_type=_f32)
    as2 = jnp.dot(h2, as2_ref[...], preferred_element_type=_f32)
    ad2 = jnp.dot(h2, ad2_ref[...], preferred_element_type=_f32)
    zero7 = jnp.zeros((BLK, 7), _f32)
    h2p_ref[0] = jnp.concatenate([h2[:, :64], as2, zero7], axis=1)
    h2p_ref[1] = jnp.concatenate([h2[:, 64:], as2, zero7], axis=1)
    t2d_ref[...] = jnp.concatenate([ad2, zero7], axis=1)
    cur = jnp.full((8, 8), jnp.max(as2), _f32)

    @pl.when(i == 0)
    def _():
        a2_ref[...] = cur

    @pl.when(i > 0)
    def _():
        a2_ref[...] = jnp.maximum(a2_ref[...], cur)


def _call_c(oa1, b1r, w2, as2c, ad2c, r):
    return pl.pallas_call(
        _tc_c,
        grid=(NB,),
        in_specs=[
            pl.BlockSpec((BLK, K1), lambda i: (i, 0)),
            pl.BlockSpec((BLK, K1), lambda i: (i + NB, 0)),
            pl.BlockSpec((1, 256), lambda i: (0, 0)),
            pl.BlockSpec((256, 128), lambda i: (0, 0)),
            pl.BlockSpec((128, 1), lambda i: (0, 0)),
            pl.BlockSpec((128, 1), lambda i: (0, 0)),
            pl.BlockSpec((4, 128), lambda i: (0, 0)),
        ],
        out_specs=[
            pl.BlockSpec((2, BLK, K2), lambda i: (0, i, 0)),
            pl.BlockSpec((BLK, 8), lambda i: (i, 0)),
            pl.BlockSpec((8, 8), lambda i: (0, 0)),
        ],
        out_shape=[
            jax.ShapeDtypeStruct((2, N, K2), _f32),
            jax.ShapeDtypeStruct((N, 8), _f32),
            jax.ShapeDtypeStruct((8, 8), _f32),
        ],
    )(oa1, oa1, b1r, w2, as2c, ad2c, r)


def _tc_e(oa0_ref, oa1_ref, b2_ref, out_ref):
    f0 = oa0_ref[...]
    f1 = oa1_ref[...]
    v0 = f0[:, :64] / (f0[:, 64:65] + 1e-16)
    v1 = f1[:, :64] / (f1[:, 64:65] + 1e-16)
    v = jnp.concatenate([v0, v1], axis=1) + b2_ref[...]
    m = jnp.max(v, axis=1, keepdims=True)
    z = v - m
    out_ref[...] = z - jnp.log(jnp.sum(jnp.exp(z), axis=1, keepdims=True))


def _call_e(oa2, b2r):
    return pl.pallas_call(
        _tc_e,
        grid=(NB,),
        in_specs=[
            pl.BlockSpec((BLK, K2), lambda i: (i, 0)),
            pl.BlockSpec((BLK, K2), lambda i: (i + NB, 0)),
            pl.BlockSpec((1, 128), lambda i: (0, 0)),
        ],
        out_specs=pl.BlockSpec((BLK, 128), lambda i: (i, 0)),
        out_shape=jax.ShapeDtypeStruct((N, 128), _f32),
    )(oa2, oa2, b2r)


def kernel(x, edge_index, W1, a_src1, a_dst1, b1, W2, a_src2, a_dst2, b2):
    rows = jnp.arange(HEADS * HID)
    hmask = (rows[:, None] // HID) == jnp.arange(8)[None, :]
    asm = jnp.where(hmask, a_src1.reshape(-1)[:, None], 0.0).astype(_f32)
    adm = jnp.where(hmask, a_dst1.reshape(-1)[:, None], 0.0).astype(_f32)
    r = (jnp.arange(128)[None, :] // 32 == jnp.arange(4)[:, None]).astype(_f32)

    srcp = jnp.pad(edge_index[0].reshape(NSUB, EDGES_PER_SUB),
                   ((0, 0), (0, ESUB - EDGES_PER_SUB)))
    dstp = jnp.pad(edge_index[1].reshape(NSUB, EDGES_PER_SUB),
                   ((0, 0), (0, ESUB - EDGES_PER_SUB)),
                   constant_values=N)
    eip = jnp.stack([srcp, dstp])
    ei4_1 = eip.reshape(2, NSUB, ESUB // EB1, EB1)
    ei4_2 = eip.reshape(2, NSUB, ESUB // EB2, EB2)
    pad8 = jnp.zeros((8, 8), _f32)

    haug, t1d, a1 = _call_a(x, W1, asm, adm)
    h1flat = haug.reshape(2 * N, K1)
    asmax1 = a1[0]
    a4, b4 = asmax1[:4], asmax1[4:]
    asm1v = jnp.stack([jnp.concatenate([a4, a4, a4, a4]),
                       jnp.concatenate([b4, b4, b4, b4])])

    oa1 = _call_sc(_sc_l1, K1, EB1, ei4_1,
                   jnp.concatenate([t1d, pad8], axis=0), h1flat, asm1v)

    h2p, t2d, a2 = _call_c(oa1, b1.reshape(1, 256), W2,
                           a_src2.reshape(128, 1), a_dst2.reshape(128, 1), r)
    h2flat = h2p.reshape(2 * N, K2)
    asm2v = jnp.full((2, 16), a2[0, 0], _f32)

    oa2 = _call_sc(_sc_l2, K2, EB2, ei4_2,
                   jnp.concatenate([t2d, pad8], axis=0), h2flat, asm2v)
    return _call_e(oa2, b2.reshape(1, 128))


# speedup vs baseline: 1.0136x; 1.0136x over previous
import functools

import jax
import jax.numpy as jnp
from jax import lax
from jax.experimental import pallas as pl
from jax.experimental.pallas import tpu as pltpu
from jax.experimental.pallas import tpu_sc as plsc

N = 10000
E = 320000
D_IN = 128
HID = 32
HEADS = 8
D_OUT = 128

BLK = 1000
NB = N // BLK
NSUB = 16
NCORE = 2
EDGES_PER_SUB = E // NSUB
ESUB = 20224
EB1 = 64
NPAIR1 = ESUB // (2 * EB1)
EB2 = 128
NPAIR2 = ESUB // (2 * EB2)
NSH = N + 8
ROWS_PER_SUB = 640
K1 = 136
K2 = 72

_f32 = jnp.float32
_i32 = jnp.int32

_GDN = lax.GatherDimensionNumbers(
    offset_dims=(), collapsed_slice_dims=(0,), start_index_map=(0,))


def _bcast(vec, idx):
    return lax.gather(vec, idx[:, None], _GDN, (1,),
                      mode=lax.GatherScatterMode.PROMISE_IN_BOUNDS)


def _tc_a(x_ref, w1_ref, asm_ref, adm_ref, haug_ref, t1d_ref, a1_ref):
    i = pl.program_id(0)
    h = jnp.dot(x_ref[...], w1_ref[...], preferred_element_type=_f32)
    als = jnp.dot(h, asm_ref[...], preferred_element_type=_f32)
    ald = jnp.dot(h, adm_ref[...], preferred_element_type=_f32)
    haug_ref[0] = jnp.concatenate([h[:, :128], als], axis=1)
    haug_ref[1] = jnp.concatenate([h[:, 128:], als], axis=1)
    t1d_ref[...] = ald
    cur = jnp.broadcast_to(jnp.max(als, axis=0)[None, :], (8, 8))

    @pl.when(i == 0)
    def _():
        a1_ref[...] = cur

    @pl.when(i > 0)
    def _():
        a1_ref[...] = jnp.maximum(a1_ref[...], cur)


def _call_a(x, w1, asm, adm):
    return pl.pallas_call(
        _tc_a,
        grid=(NB,),
        in_specs=[
            pl.BlockSpec((BLK, D_IN), lambda i: (i, 0)),
            pl.BlockSpec((D_IN, HEADS * HID), lambda i: (0, 0)),
            pl.BlockSpec((HEADS * HID, 8), lambda i: (0, 0)),
            pl.BlockSpec((HEADS * HID, 8), lambda i: (0, 0)),
        ],
        out_specs=[
            pl.BlockSpec((2, BLK, K1), lambda i: (0, i, 0)),
            pl.BlockSpec((BLK, 8), lambda i: (i, 0)),
            pl.BlockSpec((8, 8), lambda i: (0, 0)),
        ],
        out_shape=[
            jax.ShapeDtypeStruct((2, N, K1), _f32),
            jax.ShapeDtypeStruct((N, 8), _f32),
            jax.ShapeDtypeStruct((8, 8), _f32),
        ],
    )(x, w1, asm, adm)


_MESH = plsc.VectorSubcoreMesh(core_axis_name="c", subcore_axis_name="s")


def _zero_msg(msg, EB, K, full):
    zero16 = jnp.zeros((16,), _f32)
    for r in range(EB):
        if full:
            for k in range(K // 16):
                msg[r, pl.ds(k * 16, 16)] = zero16
        msg[r, pl.ds(K - 16, 16)] = zero16


def _zero_stripe(msg, shared, s, base_r, EB):
    n0 = ROWS_PER_SUB // EB
    n15 = 400 // EB
    tail = 400 - n15 * EB

    @pl.when(s < 15)
    def _():
        for j in range(n0):
            pltpu.sync_copy(msg, shared.at[pl.ds(base_r + j * EB, EB)])

    @pl.when(s == 15)
    def _():
        for j in range(n15):
            pltpu.sync_copy(msg, shared.at[pl.ds(base_r + j * EB, EB)])
        if tail:
            pltpu.sync_copy(msg.at[pl.ds(0, tail)],
                            shared.at[pl.ds(base_r + n15 * EB, tail)])


def _copy_out(msg, shared, oa, s, base_r, out_base, EB):
    n0 = ROWS_PER_SUB // EB
    n15 = 400 // EB
    tail = 400 - n15 * EB

    @pl.when(s < 15)
    def _():
        for j in range(n0):
            pltpu.sync_copy(shared.at[pl.ds(base_r + j * EB, EB)], msg)
            pltpu.sync_copy(msg, oa.at[pl.ds(out_base + j * EB, EB)])

    @pl.when(s == 15)
    def _():
        for j in range(n15):
            pltpu.sync_copy(shared.at[pl.ds(base_r + j * EB, EB)], msg)
            pltpu.sync_copy(msg, oa.at[pl.ds(out_base + j * EB, EB)])
        if tail:
            pltpu.sync_copy(shared.at[pl.ds(base_r + n15 * EB, tail)],
                            msg.at[pl.ds(0, tail)])
            pltpu.sync_copy(msg.at[pl.ds(0, tail)],
                            oa.at[pl.ds(out_base + n15 * EB, tail)])


def _make_sc_edge(NH, F, K, EB, NPAIR, as_col_fn, ad_col_fn):
    CPH = F // NH // 16
    if EB == 64:
        def bqoff(i, p):
            return i + (p >> 1), (p & 1) * EB
    else:
        def bqoff(i, p):
            return 2 * i + p, 0

    def kern(ei, tt, hh, asmv, oa, shared, srcba, srcbb, dstba, dstbb,
             sdsta, sdstb, offba, offbb, tdsta, tdstb, hbufa, hbufb,
             msga, msgb, asmb, isema, isemb, gsema, gsemb, ssema, ssemb):
        c = lax.axis_index("c")
        s = lax.axis_index("s")
        lane = lax.iota(_i32, 16)
        pltpu.sync_copy(asmv.at[c], asmb)

        _zero_msg(msga, EB, K, full=True)
        _zero_msg(msgb, EB, K, full=False)
        base_r = s * ROWS_PER_SUB
        _zero_stripe(msga, shared, s, base_r, EB)
        plsc.subcore_barrier()

        cn = c * N
        asmlv = asmb[...]
        asml = [_bcast(asmlv, jnp.full((16,), h, _i32)) for h in range(NH)]
        cas = [as_col_fn(h, c) for h in range(NH)]
        cad = [ad_col_fn(h, c) for h in range(NH)]

        def enq_idx(bq, off, srcb, dstb, isem):
            pltpu.async_copy(ei.at[0, s, bq, pl.ds(off, EB)], srcb, isem)
            pltpu.async_copy(ei.at[1, s, bq, pl.ds(off, EB)], dstb, isem)

        def wait_idx(bq, off, srcb, dstb, isem):
            pltpu.make_async_copy(
                ei.at[0, s, bq, pl.ds(off, EB)], srcb, isem).wait()
            pltpu.make_async_copy(
                ei.at[1, s, bq, pl.ds(off, EB)], dstb, isem).wait()

        def fill_off(srcb, offb):
            for j in range(EB // 16):
                offb[pl.ds(j * 16, 16)] = srcb[pl.ds(j * 16, 16)] + cn

        def enq_g(offb, dstb, hbuf, tdst, gsem):
            pltpu.async_copy(hh.at[offb], hbuf, gsem)
            pltpu.async_copy(tt.at[dstb], tdst, gsem)

        def wait_g(offb, dstb, hbuf, tdst, gsem):
            pltpu.make_async_copy(hh.at[offb], hbuf, gsem).wait()
            pltpu.make_async_copy(tt.at[dstb], tdst, gsem).wait()

        def fill_sdst(dstb, sdst):
            for j in range(EB // 16):
                sdst[pl.ds(j * 16, 16)] = dstb[pl.ds(j * 16, 16)]

        def enq_s(msg, sdst, ssem):
            pltpu.async_copy(msg, shared.at[sdst], ssem, add=True)

        def wait_s(msg, sdst, ssem):
            pltpu.make_async_copy(msg, shared.at[sdst], ssem).wait()

        def compute(hbuf, tdst, msg):
            for j in range(EB // 16):
                rowv = lane + j * 16
                ees = []
                for h in range(NH):
                    asv = plsc.load_gather(hbuf, [rowv, cas[h]])
                    adv = plsc.load_gather(tdst, [rowv, cad[h]])
                    sv = asv + adv
                    ev = jnp.where(sv > 0, sv, 0.2 * sv)
                    mr = asml[h] + adv
                    mv = jnp.where(mr > 0, mr, 0.2 * mr)
                    ees.append(jnp.exp(ev - mv))
                for h in range(NH):
                    plsc.store_scatter(
                        msg, [rowv, jnp.full((16,), F + h, _i32)], ees[h])
                for e in range(16):
                    row = j * 16 + e
                    idx = jnp.full((16,), e, _i32)
                    for h in range(NH):
                        b = _bcast(ees[h], idx)
                        for k2 in range(CPH):
                            col = (h * CPH + k2) * 16
                            msg[row, pl.ds(col, 16)] = (
                                hbuf[row, pl.ds(col, 16)] * b)

        enq_idx(*bqoff(0, 0), srcba, dstba, isema)
        enq_idx(*bqoff(0, 1), srcbb, dstbb, isemb)
        wait_idx(*bqoff(0, 0), srcba, dstba, isema)
        fill_off(srcba, offba)
        enq_g(offba, dstba, hbufa, tdsta, gsema)

        def body(i, _):
            wait_idx(*bqoff(i, 1), srcbb, dstbb, isemb)
            fill_off(srcbb, offbb)
            enq_g(offbb, dstbb, hbufb, tdstb, gsemb)
            wait_g(offba, dstba, hbufa, tdsta, gsema)

            @pl.when(i > 0)
            def _():
                wait_s(msga, sdsta, ssema)

            fill_sdst(dstba, sdsta)

            @pl.when(i < NPAIR - 1)
            def _():
                enq_idx(*bqoff(i, 2), srcba, dstba, isema)

            compute(hbufa, tdsta, msga)
            enq_s(msga, sdsta, ssema)

            @pl.when(i < NPAIR - 1)
            def _():
                wait_idx(*bqoff(i, 2), srcba, dstba, isema)
                fill_off(srcba, offba)
                enq_g(offba, dstba, hbufa, tdsta, gsema)

            wait_g(offbb, dstbb, hbufb, tdstb, gsemb)

            @pl.when(i > 0)
            def _():
                wait_s(msgb, sdstb, ssemb)

            fill_sdst(dstbb, sdstb)

            @pl.when(i < NPAIR - 1)
            def _():
                enq_idx(*bqoff(i, 3), srcbb, dstbb, isemb)

            compute(hbufb, tdstb, msgb)
            enq_s(msgb, sdstb, ssemb)
            return 0

        lax.fori_loop(0, NPAIR, body, 0)
        wait_s(msga, sdsta, ssema)
        wait_s(msgb, sdstb, ssemb)
        plsc.subcore_barrier()
        _copy_out(msga, shared, oa, s, base_r, cn + base_r, EB)

    return kern


def _call_sc(kern, K, EB, ei4, tt, hh, asmv):
    fn = functools.partial(
        pl.kernel,
        out_type=jax.ShapeDtypeStruct((2 * N, K), _f32),
        mesh=_MESH,
        compiler_params=pltpu.CompilerParams(
            use_tc_tiling_on_sc=False, needs_layout_passes=False),
        scratch_types=[
            pltpu.VMEM_SHARED((NSH, K), _f32),
            pltpu.VMEM((EB,), _i32),
            pltpu.VMEM((EB,), _i32),
            pltpu.VMEM((EB,), _i32),
            pltpu.VMEM((EB,), _i32),
            pltpu.VMEM((EB,), _i32),
            pltpu.VMEM((EB,), _i32),
            pltpu.VMEM((EB,), _i32),
            pltpu.VMEM((EB,), _i32),
            pltpu.VMEM((EB, 8), _f32),
            pltpu.VMEM((EB, 8), _f32),
            pltpu.VMEM((EB, K), _f32),
            pltpu.VMEM((EB, K), _f32),
            pltpu.VMEM((EB, K), _f32),
            pltpu.VMEM((EB, K), _f32),
            pltpu.VMEM((16,), _f32),

            pltpu.SemaphoreType.DMA,
            pltpu.SemaphoreType.DMA,
            pltpu.SemaphoreType.DMA,
            pltpu.SemaphoreType.DMA,
            pltpu.SemaphoreType.DMA,
            pltpu.SemaphoreType.DMA,
        ],
    )(kern)
    return fn(ei4, tt, hh, asmv)


_sc_l1 = _make_sc_edge(
    4, 128, K1, EB1, NPAIR1,
    lambda h, c: jnp.full((16,), 128 + h, _i32) + 4 * c,
    lambda h, c: jnp.full((16,), h, _i32) + 4 * c)

_sc_l2 = _make_sc_edge(
    1, 64, K2, EB2, NPAIR2,
    lambda h, c: jnp.full((16,), 64, _i32),
    lambda h, c: jnp.zeros((16,), _i32))


def _tc_c(oa0_ref, oa1_ref, b1_ref, w2_ref, as2_ref, ad2_ref, r_ref,
          h2p_ref, t2d_ref, a2_ref):
    i = pl.program_id(0)
    f0 = oa0_ref[...]
    f1 = oa1_ref[...]
    r = r_ref[...]
    den0 = jnp.dot(f0[:, 128:132], r, preferred_element_type=_f32)
    den1 = jnp.dot(f1[:, 128:132], r, preferred_element_type=_f32)
    p0 = f0[:, :128] / (den0 + 1e-16)
    p1 = f1[:, :128] / (den1 + 1e-16)
    h1n = jnp.concatenate([p0, p1], axis=1) + b1_ref[...]
    e1 = jnp.where(h1n > 0, h1n, jnp.exp(h1n) - 1.0)
    h2 = jnp.dot(e1, w2_ref[...], preferred_element_type=_f32)
    as2 = jnp.dot(h2, as2_ref[...], preferred_element_type=_f32)
    ad2 = jnp.dot(h2, ad2_ref[...], preferred_element_type=_f32)
    zero7 = jnp.zeros((BLK, 7), _f32)
    h2p_ref[0] = jnp.concatenate([h2[:, :64], as2, zero7], axis=1)
    h2p_ref[1] = jnp.concatenate([h2[:, 64:], as2, zero7], axis=1)
    t2d_ref[...] = jnp.concatenate([ad2, zero7], axis=1)
    cur = jnp.full((8, 8), jnp.max(as2), _f32)

    @pl.when(i == 0)
    def _():
        a2_ref[...] = cur

    @pl.when(i > 0)
    def _():
        a2_ref[...] = jnp.maximum(a2_ref[...], cur)


def _call_c(oa1, b1r, w2, as2c, ad2c, r):
    return pl.pallas_call(
        _tc_c,
        grid=(NB,),
        in_specs=[
            pl.BlockSpec((BLK, K1), lambda i: (i, 0)),
            pl.BlockSpec((BLK, K1), lambda i: (i + NB, 0)),
            pl.BlockSpec((1, 256), lambda i: (0, 0)),
            pl.BlockSpec((256, 128), lambda i: (0, 0)),
            pl.BlockSpec((128, 1), lambda i: (0, 0)),
            pl.BlockSpec((128, 1), lambda i: (0, 0)),
            pl.BlockSpec((4, 128), lambda i: (0, 0)),
        ],
        out_specs=[
            pl.BlockSpec((2, BLK, K2), lambda i: (0, i, 0)),
            pl.BlockSpec((BLK, 8), lambda i: (i, 0)),
            pl.BlockSpec((8, 8), lambda i: (0, 0)),
        ],
        out_shape=[
            jax.ShapeDtypeStruct((2, N, K2), _f32),
            jax.ShapeDtypeStruct((N, 8), _f32),
            jax.ShapeDtypeStruct((8, 8), _f32),
        ],
    )(oa1, oa1, b1r, w2, as2c, ad2c, r)


def _tc_e(oa0_ref, oa1_ref, b2_ref, out_ref):
    f0 = oa0_ref[...]
    f1 = oa1_ref[...]
    v0 = f0[:, :64] / (f0[:, 64:65] + 1e-16)
    v1 = f1[:, :64] / (f1[:, 64:65] + 1e-16)
    v = jnp.concatenate([v0, v1], axis=1) + b2_ref[...]
    m = jnp.max(v, axis=1, keepdims=True)
    z = v - m
    out_ref[...] = z - jnp.log(jnp.sum(jnp.exp(z), axis=1, keepdims=True))


def _call_e(oa2, b2r):
    return pl.pallas_call(
        _tc_e,
        grid=(NB,),
        in_specs=[
            pl.BlockSpec((BLK, K2), lambda i: (i, 0)),
            pl.BlockSpec((BLK, K2), lambda i: (i + NB, 0)),
            pl.BlockSpec((1, 128), lambda i: (0, 0)),
        ],
        out_specs=pl.BlockSpec((BLK, 128), lambda i: (i, 0)),
        out_shape=jax.ShapeDtypeStruct((N, 128), _f32),
    )(oa2, oa2, b2r)


def kernel(x, edge_index, W1, a_src1, a_dst1, b1, W2, a_src2, a_dst2, b2):
    rows = jnp.arange(HEADS * HID)
    hmask = (rows[:, None] // HID) == jnp.arange(8)[None, :]
    asm = jnp.where(hmask, a_src1.reshape(-1)[:, None], 0.0).astype(_f32)
    adm = jnp.where(hmask, a_dst1.reshape(-1)[:, None], 0.0).astype(_f32)
    r = (jnp.arange(128)[None, :] // 32 == jnp.arange(4)[:, None]).astype(_f32)

    srcp = jnp.pad(edge_index[0].reshape(NSUB, EDGES_PER_SUB),
                   ((0, 0), (0, ESUB - EDGES_PER_SUB)))
    dstp = jnp.pad(edge_index[1].reshape(NSUB, EDGES_PER_SUB),
                   ((0, 0), (0, ESUB - EDGES_PER_SUB)),
                   constant_values=N)
    eip = jnp.stack([srcp, dstp])
    ei4 = eip.reshape(2, NSUB, ESUB // EB2, EB2)
    pad8 = jnp.zeros((8, 8), _f32)

    haug, t1d, a1 = _call_a(x, W1, asm, adm)
    h1flat = haug.reshape(2 * N, K1)
    asmax1 = a1[0]
    a4, b4 = asmax1[:4], asmax1[4:]
    asm1v = jnp.stack([jnp.concatenate([a4, a4, a4, a4]),
                       jnp.concatenate([b4, b4, b4, b4])])

    oa1 = _call_sc(_sc_l1, K1, EB1, ei4,
                   jnp.concatenate([t1d, pad8], axis=0), h1flat, asm1v)

    h2p, t2d, a2 = _call_c(oa1, b1.reshape(1, 256), W2,
                           a_src2.reshape(128, 1), a_dst2.reshape(128, 1), r)
    h2flat = h2p.reshape(2 * N, K2)
    asm2v = jnp.full((2, 16), a2[0, 0], _f32)

    oa2 = _call_sc(_sc_l2, K2, EB2, ei4,
                   jnp.concatenate([t2d, pad8], axis=0), h2flat, asm2v)
    return _call_e(oa2, b2.reshape(1, 128))


# speedup vs baseline: 1.1027x; 1.0880x over previous
import functools

import jax
import jax.numpy as jnp
from jax import lax
from jax.experimental import pallas as pl
from jax.experimental.pallas import tpu as pltpu
from jax.experimental.pallas import tpu_sc as plsc

N = 10000
E = 320000
D_IN = 128
HID = 32
HEADS = 8
D_OUT = 128

BLK = 1000
NB = N // BLK
NSUB = 16
NCORE = 2
EDGES_PER_SUB = E // NSUB
EB1 = 64
ESUB1 = 20096
NPAIR1 = ESUB1 // (2 * EB1)
EB2 = 128
ESUB2 = 20224
NPAIR2 = ESUB2 // (2 * EB2)
NSH = N + 8
ROWS_PER_SUB = 640
K1 = 136
K2 = 72

_f32 = jnp.float32
_i32 = jnp.int32

_GDN = lax.GatherDimensionNumbers(
    offset_dims=(), collapsed_slice_dims=(0,), start_index_map=(0,))


def _bcast(vec, idx):
    return lax.gather(vec, idx[:, None], _GDN, (1,),
                      mode=lax.GatherScatterMode.PROMISE_IN_BOUNDS)


def _tc_a(x_ref, w1_ref, asm_ref, adm_ref, haug_ref, t1d_ref, a1_ref):
    i = pl.program_id(0)
    h = jnp.dot(x_ref[...], w1_ref[...], preferred_element_type=_f32)
    als = jnp.dot(h, asm_ref[...], preferred_element_type=_f32)
    ald = jnp.dot(h, adm_ref[...], preferred_element_type=_f32)
    haug_ref[0] = jnp.concatenate([h[:, :128], als], axis=1)
    haug_ref[1] = jnp.concatenate([h[:, 128:], als], axis=1)
    t1d_ref[...] = ald
    cur = jnp.broadcast_to(jnp.max(als, axis=0)[None, :], (8, 8))

    @pl.when(i == 0)
    def _():
        a1_ref[...] = cur

    @pl.when(i > 0)
    def _():
        a1_ref[...] = jnp.maximum(a1_ref[...], cur)


def _call_a(x, w1, asm, adm):
    return pl.pallas_call(
        _tc_a,
        grid=(NB,),
        in_specs=[
            pl.BlockSpec((BLK, D_IN), lambda i: (i, 0)),
            pl.BlockSpec((D_IN, HEADS * HID), lambda i: (0, 0)),
            pl.BlockSpec((HEADS * HID, 8), lambda i: (0, 0)),
            pl.BlockSpec((HEADS * HID, 8), lambda i: (0, 0)),
        ],
        out_specs=[
            pl.BlockSpec((2, BLK, K1), lambda i: (0, i, 0)),
            pl.BlockSpec((BLK, 8), lambda i: (i, 0)),
            pl.BlockSpec((8, 8), lambda i: (0, 0)),
        ],
        out_shape=[
            jax.ShapeDtypeStruct((2, N, K1), _f32),
            jax.ShapeDtypeStruct((N, 8), _f32),
            jax.ShapeDtypeStruct((8, 8), _f32),
        ],
    )(x, w1, asm, adm)


_MESH = plsc.VectorSubcoreMesh(core_axis_name="c", subcore_axis_name="s")


def _zero_msg(msg, EB, K, full):
    zero16 = jnp.zeros((16,), _f32)
    for r in range(EB):
        if full:
            for k in range(K // 16):
                msg[r, pl.ds(k * 16, 16)] = zero16
        msg[r, pl.ds(K - 16, 16)] = zero16


def _zero_stripe(msg, shared, s, base_r, EB):
    n0 = ROWS_PER_SUB // EB
    n15 = 400 // EB
    tail = 400 - n15 * EB

    @pl.when(s < 15)
    def _():
        for j in range(n0):
            pltpu.sync_copy(msg, shared.at[pl.ds(base_r + j * EB, EB)])

    @pl.when(s == 15)
    def _():
        for j in range(n15):
            pltpu.sync_copy(msg, shared.at[pl.ds(base_r + j * EB, EB)])
        if tail:
            pltpu.sync_copy(msg.at[pl.ds(0, tail)],
                            shared.at[pl.ds(base_r + n15 * EB, tail)])


def _copy_out(msg, shared, oa, s, base_r, out_base, EB):
    n0 = ROWS_PER_SUB // EB
    n15 = 400 // EB
    tail = 400 - n15 * EB

    @pl.when(s < 15)
    def _():
        for j in range(n0):
            pltpu.sync_copy(shared.at[pl.ds(base_r + j * EB, EB)], msg)
            pltpu.sync_copy(msg, oa.at[pl.ds(out_base + j * EB, EB)])

    @pl.when(s == 15)
    def _():
        for j in range(n15):
            pltpu.sync_copy(shared.at[pl.ds(base_r + j * EB, EB)], msg)
            pltpu.sync_copy(msg, oa.at[pl.ds(out_base + j * EB, EB)])
        if tail:
            pltpu.sync_copy(shared.at[pl.ds(base_r + n15 * EB, tail)],
                            msg.at[pl.ds(0, tail)])
            pltpu.sync_copy(msg.at[pl.ds(0, tail)],
                            oa.at[pl.ds(out_base + n15 * EB, tail)])


def _make_sc_edge(NH, F, K, EB, NPAIR, as_col_fn, ad_col_fn):
    CPH = F // NH // 16

    def bqoff(i, p):
        return 2 * i + p, 0

    def kern(ei, tt, hh, asmv, oa, shared, srcba, srcbb, dstba, dstbb,
             sdsta, sdstb, offba, offbb, tdsta, tdstb, hbufa, hbufb,
             msga, msgb, asmb, isema, isemb, gsema, gsemb, ssema, ssemb):
        c = lax.axis_index("c")
        s = lax.axis_index("s")
        lane = lax.iota(_i32, 16)
        pltpu.sync_copy(asmv.at[c], asmb)

        _zero_msg(msga, EB, K, full=True)
        _zero_msg(msgb, EB, K, full=False)
        base_r = s * ROWS_PER_SUB
        _zero_stripe(msga, shared, s, base_r, EB)
        plsc.subcore_barrier()

        cn = c * N
        asmlv = asmb[...]
        asml = [_bcast(asmlv, jnp.full((16,), h, _i32)) for h in range(NH)]
        cas = [as_col_fn(h, c) for h in range(NH)]
        cad = [ad_col_fn(h, c) for h in range(NH)]

        def enq_idx(bq, off, srcb, dstb, isem):
            pltpu.async_copy(ei.at[0, s, bq, pl.ds(off, EB)], srcb, isem)
            pltpu.async_copy(ei.at[1, s, bq, pl.ds(off, EB)], dstb, isem)

        def wait_idx(bq, off, srcb, dstb, isem):
            pltpu.make_async_copy(
                ei.at[0, s, bq, pl.ds(off, EB)], srcb, isem).wait()
            pltpu.make_async_copy(
                ei.at[1, s, bq, pl.ds(off, EB)], dstb, isem).wait()

        def fill_off(srcb, offb):
            for j in range(EB // 16):
                offb[pl.ds(j * 16, 16)] = srcb[pl.ds(j * 16, 16)] + cn

        def enq_g(offb, dstb, hbuf, tdst, gsem):
            pltpu.async_copy(hh.at[offb], hbuf, gsem)
            pltpu.async_copy(tt.at[dstb], tdst, gsem)

        def wait_g(offb, dstb, hbuf, tdst, gsem):
            pltpu.make_async_copy(hh.at[offb], hbuf, gsem).wait()
            pltpu.make_async_copy(tt.at[dstb], tdst, gsem).wait()

        def fill_sdst(dstb, sdst):
            for j in range(EB // 16):
                sdst[pl.ds(j * 16, 16)] = dstb[pl.ds(j * 16, 16)]

        def enq_s(msg, sdst, ssem):
            pltpu.async_copy(msg, shared.at[sdst], ssem, add=True)

        def wait_s(msg, sdst, ssem):
            pltpu.make_async_copy(msg, shared.at[sdst], ssem).wait()

        def compute(hbuf, tdst, msg):
            for j in range(EB // 16):
                rowv = lane + j * 16
                ees = []
                for h in range(NH):
                    asv = plsc.load_gather(hbuf, [rowv, cas[h]])
                    adv = plsc.load_gather(tdst, [rowv, cad[h]])
                    sv = asv + adv
                    ev = jnp.where(sv > 0, sv, 0.2 * sv)
                    mr = asml[h] + adv
                    mv = jnp.where(mr > 0, mr, 0.2 * mr)
                    ees.append(jnp.exp(ev - mv))
                for h in range(NH):
                    plsc.store_scatter(
                        msg, [rowv, jnp.full((16,), F + h, _i32)], ees[h])
                for e in range(16):
                    row = j * 16 + e
                    idx = jnp.full((16,), e, _i32)
                    for h in range(NH):
                        b = _bcast(ees[h], idx)
                        for k2 in range(CPH):
                            col = (h * CPH + k2) * 16
                            msg[row, pl.ds(col, 16)] = (
                                hbuf[row, pl.ds(col, 16)] * b)

        enq_idx(*bqoff(0, 0), srcba, dstba, isema)
        enq_idx(*bqoff(0, 1), srcbb, dstbb, isemb)
        wait_idx(*bqoff(0, 0), srcba, dstba, isema)
        fill_off(srcba, offba)
        enq_g(offba, dstba, hbufa, tdsta, gsema)

        def body(i, _):
            wait_idx(*bqoff(i, 1), srcbb, dstbb, isemb)
            fill_off(srcbb, offbb)
            enq_g(offbb, dstbb, hbufb, tdstb, gsemb)
            wait_g(offba, dstba, hbufa, tdsta, gsema)

            @pl.when(i > 0)
            def _():
                wait_s(msga, sdsta, ssema)

            fill_sdst(dstba, sdsta)

            @pl.when(i < NPAIR - 1)
            def _():
                enq_idx(*bqoff(i, 2), srcba, dstba, isema)

            compute(hbufa, tdsta, msga)
            enq_s(msga, sdsta, ssema)

            @pl.when(i < NPAIR - 1)
            def _():
                wait_idx(*bqoff(i, 2), srcba, dstba, isema)
                fill_off(srcba, offba)
                enq_g(offba, dstba, hbufa, tdsta, gsema)

            wait_g(offbb, dstbb, hbufb, tdstb, gsemb)

            @pl.when(i > 0)
            def _():
                wait_s(msgb, sdstb, ssemb)

            fill_sdst(dstbb, sdstb)

            @pl.when(i < NPAIR - 1)
            def _():
                enq_idx(*bqoff(i, 3), srcbb, dstbb, isemb)

            compute(hbufb, tdstb, msgb)
            enq_s(msgb, sdstb, ssemb)
            return 0

        lax.fori_loop(0, NPAIR, body, 0)
        wait_s(msga, sdsta, ssema)
        wait_s(msgb, sdstb, ssemb)
        plsc.subcore_barrier()
        _copy_out(msga, shared, oa, s, base_r, cn + base_r, EB)

    return kern


def _call_sc(kern, K, EB, ei4, tt, hh, asmv):
    fn = functools.partial(
        pl.kernel,
        out_type=jax.ShapeDtypeStruct((2 * N, K), _f32),
        mesh=_MESH,
        compiler_params=pltpu.CompilerParams(
            use_tc_tiling_on_sc=False, needs_layout_passes=False),
        scratch_types=[
            pltpu.VMEM_SHARED((NSH, K), _f32),
            pltpu.VMEM((EB,), _i32),
            pltpu.VMEM((EB,), _i32),
            pltpu.VMEM((EB,), _i32),
            pltpu.VMEM((EB,), _i32),
            pltpu.VMEM((EB,), _i32),
            pltpu.VMEM((EB,), _i32),
            pltpu.VMEM((EB,), _i32),
            pltpu.VMEM((EB,), _i32),
            pltpu.VMEM((EB, 8), _f32),
            pltpu.VMEM((EB, 8), _f32),
            pltpu.VMEM((EB, K), _f32),
            pltpu.VMEM((EB, K), _f32),
            pltpu.VMEM((EB, K), _f32),
            pltpu.VMEM((EB, K), _f32),
            pltpu.VMEM((16,), _f32),

            pltpu.SemaphoreType.DMA,
            pltpu.SemaphoreType.DMA,
            pltpu.SemaphoreType.DMA,
            pltpu.SemaphoreType.DMA,
            pltpu.SemaphoreType.DMA,
            pltpu.SemaphoreType.DMA,
        ],
    )(kern)
    return fn(ei4, tt, hh, asmv)


_sc_l1 = _make_sc_edge(
    4, 128, K1, EB1, NPAIR1,
    lambda h, c: jnp.full((16,), 128 + h, _i32) + 4 * c,
    lambda h, c: jnp.full((16,), h, _i32) + 4 * c)

_sc_l2 = _make_sc_edge(
    1, 64, K2, EB2, NPAIR2,
    lambda h, c: jnp.full((16,), 64, _i32),
    lambda h, c: jnp.zeros((16,), _i32))


def _tc_c(oa0_ref, oa1_ref, b1_ref, w2_ref, as2_ref, ad2_ref, r_ref,
          h2p_ref, t2d_ref, a2_ref):
    i = pl.program_id(0)
    f0 = oa0_ref[...]
    f1 = oa1_ref[...]
    r = r_ref[...]
    den0 = jnp.dot(f0[:, 128:132], r, preferred_element_type=_f32)
    den1 = jnp.dot(f1[:, 128:132], r, preferred_element_type=_f32)
    p0 = f0[:, :128] / (den0 + 1e-16)
    p1 = f1[:, :128] / (den1 + 1e-16)
    h1n = jnp.concatenate([p0, p1], axis=1) + b1_ref[...]
    e1 = jnp.where(h1n > 0, h1n, jnp.exp(h1n) - 1.0)
    h2 = jnp.dot(e1, w2_ref[...], preferred_element_type=_f32)
    as2 = jnp.dot(h2, as2_ref[...], preferred_element_type=_f32)
    ad2 = jnp.dot(h2, ad2_ref[...], preferred_element_type=_f32)
    zero7 = jnp.zeros((BLK, 7), _f32)
    h2p_ref[0] = jnp.concatenate([h2[:, :64], as2, zero7], axis=1)
    h2p_ref[1] = jnp.concatenate([h2[:, 64:], as2, zero7], axis=1)
    t2d_ref[...] = jnp.concatenate([ad2, zero7], axis=1)
    cur = jnp.full((8, 8), jnp.max(as2), _f32)

    @pl.when(i == 0)
    def _():
        a2_ref[...] = cur

    @pl.when(i > 0)
    def _():
        a2_ref[...] = jnp.maximum(a2_ref[...], cur)


def _call_c(oa1, b1r, w2, as2c, ad2c, r):
    return pl.pallas_call(
        _tc_c,
        grid=(NB,),
        in_specs=[
            pl.BlockSpec((BLK, K1), lambda i: (i, 0)),
            pl.BlockSpec((BLK, K1), lambda i: (i + NB, 0)),
            pl.BlockSpec((1, 256), lambda i: (0, 0)),
            pl.BlockSpec((256, 128), lambda i: (0, 0)),
            pl.BlockSpec((128, 1), lambda i: (0, 0)),
            pl.BlockSpec((128, 1), lambda i: (0, 0)),
            pl.BlockSpec((4, 128), lambda i: (0, 0)),
        ],
        out_specs=[
            pl.BlockSpec((2, BLK, K2), lambda i: (0, i, 0)),
            pl.BlockSpec((BLK, 8), lambda i: (i, 0)),
            pl.BlockSpec((8, 8), lambda i: (0, 0)),
        ],
        out_shape=[
            jax.ShapeDtypeStruct((2, N, K2), _f32),
            jax.ShapeDtypeStruct((N, 8), _f32),
            jax.ShapeDtypeStruct((8, 8), _f32),
        ],
    )(oa1, oa1, b1r, w2, as2c, ad2c, r)


def _tc_e(oa0_ref, oa1_ref, b2_ref, out_ref):
    f0 = oa0_ref[...]
    f1 = oa1_ref[...]
    v0 = f0[:, :64] / (f0[:, 64:65] + 1e-16)
    v1 = f1[:, :64] / (f1[:, 64:65] + 1e-16)
    v = jnp.concatenate([v0, v1], axis=1) + b2_ref[...]
    m = jnp.max(v, axis=1, keepdims=True)
    z = v - m
    out_ref[...] = z - jnp.log(jnp.sum(jnp.exp(z), axis=1, keepdims=True))


def _call_e(oa2, b2r):
    return pl.pallas_call(
        _tc_e,
        grid=(NB,),
        in_specs=[
            pl.BlockSpec((BLK, K2), lambda i: (i, 0)),
            pl.BlockSpec((BLK, K2), lambda i: (i + NB, 0)),
            pl.BlockSpec((1, 128), lambda i: (0, 0)),
        ],
        out_specs=pl.BlockSpec((BLK, 128), lambda i: (i, 0)),
        out_shape=jax.ShapeDtypeStruct((N, 128), _f32),
    )(oa2, oa2, b2r)


def kernel(x, edge_index, W1, a_src1, a_dst1, b1, W2, a_src2, a_dst2, b2):
    rows = jnp.arange(HEADS * HID)
    hmask = (rows[:, None] // HID) == jnp.arange(8)[None, :]
    asm = jnp.where(hmask, a_src1.reshape(-1)[:, None], 0.0).astype(_f32)
    adm = jnp.where(hmask, a_dst1.reshape(-1)[:, None], 0.0).astype(_f32)
    r = (jnp.arange(128)[None, :] // 32 == jnp.arange(4)[:, None]).astype(_f32)

    def _pad_edges(esub, eb):
        srcp = jnp.pad(edge_index[0].reshape(NSUB, EDGES_PER_SUB),
                       ((0, 0), (0, esub - EDGES_PER_SUB)))
        dstp = jnp.pad(edge_index[1].reshape(NSUB, EDGES_PER_SUB),
                       ((0, 0), (0, esub - EDGES_PER_SUB)),
                       constant_values=N)
        return jnp.stack([srcp, dstp]).reshape(2, NSUB, esub // eb, eb)

    ei4_1 = _pad_edges(ESUB1, EB1)
    ei4_2 = _pad_edges(ESUB2, EB2)
    pad8 = jnp.zeros((8, 8), _f32)

    haug, t1d, a1 = _call_a(x, W1, asm, adm)
    h1flat = haug.reshape(2 * N, K1)
    asmax1 = a1[0]
    a4, b4 = asmax1[:4], asmax1[4:]
    asm1v = jnp.stack([jnp.concatenate([a4, a4, a4, a4]),
                       jnp.concatenate([b4, b4, b4, b4])])

    oa1 = _call_sc(_sc_l1, K1, EB1, ei4_1,
                   jnp.concatenate([t1d, pad8], axis=0), h1flat, asm1v)

    h2p, t2d, a2 = _call_c(oa1, b1.reshape(1, 256), W2,
                           a_src2.reshape(128, 1), a_dst2.reshape(128, 1), r)
    h2flat = h2p.reshape(2 * N, K2)
    asm2v = jnp.full((2, 16), a2[0, 0], _f32)

    oa2 = _call_sc(_sc_l2, K2, EB2, ei4_2,
                   jnp.concatenate([t2d, pad8], axis=0), h2flat, asm2v)
    return _call_e(oa2, b2.reshape(1, 128))
